# Initial kernel scaffold; baseline (speedup 1.0000x reference)
import jax
import jax.numpy as jnp
from jax.experimental import pallas as pl


def kernel(nodes, edges, globals_, senders, receivers, We_W, We_b, Wn_W, Wn_b, Wg_W, Wg_b):
    raise NotImplementedError("write your pallas kernel here")


# speedup vs baseline: 3.3354x; 3.3354x over previous
import functools

import jax
import jax.numpy as jnp
from jax import lax
from jax.experimental import pallas as pl
from jax.experimental.pallas import tpu as pltpu
from jax.experimental.pallas import tpu_sc as plsc

F32 = jnp.float32

_N = 10000
_E = 320000
_DN = 128
_DE = 16
_DG = 8

_CHUNK = 128
_NCHUNKS = _E // _CHUNK
_NW = 32
_NPAD = 10240
_ROWS_PER_SUB = _NPAD // 16

_NBLK = 400
_EBLK = 2000


def _init_body(nodes_ref, g_ref, wes_ref, wer_ref, weg_ref, web_ref,
               p_ref, q_ref, ce_ref):
    nb = nodes_ref[...]
    p_ref[...] = jnp.dot(nb, wes_ref[...], preferred_element_type=F32)
    q_ref[...] = jnp.dot(nb, wer_ref[...], preferred_element_type=F32)

    @pl.when(pl.program_id(0) == 0)
    def _():
        ce_ref[...] = (
            jnp.dot(g_ref[...], weg_ref[...], preferred_element_type=F32)
            + web_ref[...]
        )


def _edge_pre_body(edges_ref, wee_ref, ce_ref, ee_ref):
    ee_ref[...] = (
        jnp.dot(edges_ref[...], wee_ref[...], preferred_element_type=F32)
        + ce_ref[...]
    )


def _sc_step_body(ee_hbm, p_hbm, q_hbm, s_hbm, r_hbm, z_hbm,
                  eout_hbm, recv_hbm,
                  idx_s, idx_r, ee_v, ps_v, qs_v, out_v, acc_sh, sem):
    cid = lax.axis_index("c")
    sid = lax.axis_index("s")
    wid = sid * 2 + cid

    row0 = sid * _ROWS_PER_SUB
    pltpu.sync_copy(z_hbm.at[pl.ds(row0, _ROWS_PER_SUB)],
                    acc_sh.at[pl.ds(row0, _ROWS_PER_SUB)])
    plsc.subcore_barrier()

    def chunk_body(j, carry):
        c = wid + j * _NW
        base = c * _CHUNK
        pltpu.sync_copy(s_hbm.at[pl.ds(base, _CHUNK)], idx_s)
        pltpu.sync_copy(r_hbm.at[pl.ds(base, _CHUNK)], idx_r)
        pltpu.sync_copy(ee_hbm.at[pl.ds(base, _CHUNK)], ee_v)
        pltpu.async_copy(p_hbm.at[idx_s], ps_v, sem).wait()
        pltpu.async_copy(q_hbm.at[idx_r], qs_v, sem).wait()

        def row_body(i, a):
            out_v[i, :] = jnp.maximum(ee_v[i, :] + ps_v[i, :] + qs_v[i, :], 0.0)
            return a

        lax.fori_loop(0, _CHUNK, row_body, 0)
        pltpu.sync_copy(out_v, eout_hbm.at[pl.ds(base, _CHUNK)])
        pltpu.sync_copy(out_v, acc_sh.at[idx_r], add=True)
        return carry

    nj = (_NCHUNKS - wid + _NW - 1) // _NW
    lax.fori_loop(0, nj, chunk_body, 0)

    plsc.subcore_barrier()
    pltpu.sync_copy(acc_sh.at[pl.ds(row0, _ROWS_PER_SUB)],
                    recv_hbm.at[cid, pl.ds(row0, _ROWS_PER_SUB)])


def _node_body(nodes_ref, recv2_ref, g_ref,
               wnn_ref, wnr_ref, wng_ref, wnb_ref,
               wes_ref, wer_ref, weg_ref, web_ref,
               wgn_ref, wge_ref, wgg_ref, wgb_ref,
               nout_ref, p_ref, q_ref, gout_ref, ce_ref,
               accn_ref, acce_ref):
    i = pl.program_id(0)
    g = g_ref[...]
    c_n = jnp.dot(g, wng_ref[...], preferred_element_type=F32) + wnb_ref[...]
    recv = recv2_ref[0] + recv2_ref[1]
    out = (
        jnp.dot(nodes_ref[...], wnn_ref[...], preferred_element_type=F32)
        + jnp.dot(recv, wnr_ref[...], preferred_element_type=F32)
        + c_n
    )
    out = jnp.maximum(out, 0.0)
    nout_ref[...] = out
    p_ref[...] = jnp.dot(out, wes_ref[...], preferred_element_type=F32)
    q_ref[...] = jnp.dot(out, wer_ref[...], preferred_element_type=F32)

    @pl.when(i == 0)
    def _():
        accn_ref[...] = jnp.zeros_like(accn_ref)
        acce_ref[...] = jnp.zeros_like(acce_ref)

    accn_ref[...] += jnp.sum(out, axis=0, keepdims=True)
    acce_ref[...] += jnp.sum(recv, axis=0, keepdims=True)

    @pl.when(i == pl.num_programs(0) - 1)
    def _():
        agg_n = accn_ref[...]
        agg_e = acce_ref[...]
        g_new = (
            jnp.dot(agg_n, wgn_ref[...], preferred_element_type=F32)
            + jnp.dot(agg_e, wge_ref[...], preferred_element_type=F32)
            + jnp.dot(g, wgg_ref[...], preferred_element_type=F32)
            + wgb_ref[...]
        )
        gout_ref[...] = g_new
        ce_ref[...] = (
            jnp.dot(g_new, weg_ref[...], preferred_element_type=F32)
            + web_ref[...]
        )


def _full(i):
    return 0


def kernel(nodes, edges, globals_, senders, receivers,
           We_W, We_b, Wn_W, Wn_b, Wg_W, Wg_b):
    We_e = We_W[:_DE]
    We_s = We_W[_DE:_DE + _DN]
    We_r = We_W[_DE + _DN:_DE + 2 * _DN]
    We_g = We_W[_DE + 2 * _DN:]
    Wn_n = Wn_W[:_DN]
    Wn_r = Wn_W[_DN:_DN + _DE]
    Wn_g = Wn_W[_DN + _DE:]
    Wg_n = Wg_W[:_DN]
    Wg_e = Wg_W[_DN:_DN + _DE]
    Wg_g = Wg_W[_DN + _DE:]
    web = We_b.reshape(1, _DE)
    wnb = Wn_b.reshape(1, _DN)
    wgb = Wg_b.reshape(1, _DG)
    zeros_pad = jnp.zeros((_NPAD, _DE), F32)

    n_grid = _N // _NBLK
    e_grid = _E // _EBLK

    p0, q0, ce0 = pl.pallas_call(
        _init_body,
        grid=(n_grid,),
        in_specs=[
            pl.BlockSpec((_NBLK, _DN), lambda i: (i, 0)),
            pl.BlockSpec((1, _DG), lambda i: (0, 0)),
            pl.BlockSpec((_DN, _DE), lambda i: (0, 0)),
            pl.BlockSpec((_DN, _DE), lambda i: (0, 0)),
            pl.BlockSpec((_DG, _DE), lambda i: (0, 0)),
            pl.BlockSpec((1, _DE), lambda i: (0, 0)),
        ],
        out_specs=[
            pl.BlockSpec((_NBLK, _DE), lambda i: (i, 0)),
            pl.BlockSpec((_NBLK, _DE), lambda i: (i, 0)),
            pl.BlockSpec((1, _DE), lambda i: (0, 0)),
        ],
        out_shape=[
            jax.ShapeDtypeStruct((_N, _DE), F32),
            jax.ShapeDtypeStruct((_N, _DE), F32),
            jax.ShapeDtypeStruct((1, _DE), F32),
        ],
    )(nodes, globals_, We_s, We_r, We_g, web)

    edge_pre = pl.pallas_call(
        _edge_pre_body,
        grid=(e_grid,),
        in_specs=[
            pl.BlockSpec((_EBLK, _DE), lambda i: (i, 0)),
            pl.BlockSpec((_DE, _DE), lambda i: (0, 0)),
            pl.BlockSpec((1, _DE), lambda i: (0, 0)),
        ],
        out_specs=pl.BlockSpec((_EBLK, _DE), lambda i: (i, 0)),
        out_shape=jax.ShapeDtypeStruct((_E, _DE), F32),
    )

    sc_step = pl.kernel(
        _sc_step_body,
        out_type=[
            jax.ShapeDtypeStruct((_E, _DE), F32),
            jax.ShapeDtypeStruct((2, _NPAD, _DE), F32),
        ],
        mesh=plsc.VectorSubcoreMesh(core_axis_name="c", subcore_axis_name="s"),
        compiler_params=pltpu.CompilerParams(use_tc_tiling_on_sc=False),
        scratch_types=[
            pltpu.VMEM((_CHUNK,), jnp.int32),
            pltpu.VMEM((_CHUNK,), jnp.int32),
            pltpu.VMEM((_CHUNK, _DE), F32),
            pltpu.VMEM((_CHUNK, _DE), F32),
            pltpu.VMEM((_CHUNK, _DE), F32),
            pltpu.VMEM((_CHUNK, _DE), F32),
            pltpu.VMEM_SHARED((_NPAD, _DE), F32),
            pltpu.SemaphoreType.DMA,
        ],
    )

    node_step = pl.pallas_call(
        _node_body,
        grid=(n_grid,),
        in_specs=[
            pl.BlockSpec((_NBLK, _DN), lambda i: (i, 0)),
            pl.BlockSpec((2, _NBLK, _DE), lambda i: (0, i, 0)),
            pl.BlockSpec((1, _DG), lambda i: (0, 0)),
            pl.BlockSpec((_DN, _DN), lambda i: (0, 0)),
            pl.BlockSpec((_DE, _DN), lambda i: (0, 0)),
            pl.BlockSpec((_DG, _DN), lambda i: (0, 0)),
            pl.BlockSpec((1, _DN), lambda i: (0, 0)),
            pl.BlockSpec((_DN, _DE), lambda i: (0, 0)),
            pl.BlockSpec((_DN, _DE), lambda i: (0, 0)),
            pl.BlockSpec((_DG, _DE), lambda i: (0, 0)),
            pl.BlockSpec((1, _DE), lambda i: (0, 0)),
            pl.BlockSpec((_DN, _DG), lambda i: (0, 0)),
            pl.BlockSpec((_DE, _DG), lambda i: (0, 0)),
            pl.BlockSpec((_DG, _DG), lambda i: (0, 0)),
            pl.BlockSpec((1, _DG), lambda i: (0, 0)),
        ],
        out_specs=[
            pl.BlockSpec((_NBLK, _DN), lambda i: (i, 0)),
            pl.BlockSpec((_NBLK, _DE), lambda i: (i, 0)),
            pl.BlockSpec((_NBLK, _DE), lambda i: (i, 0)),
            pl.BlockSpec((1, _DG), lambda i: (0, 0)),
            pl.BlockSpec((1, _DE), lambda i: (0, 0)),
        ],
        out_shape=[
            jax.ShapeDtypeStruct((_N, _DN), F32),
            jax.ShapeDtypeStruct((_N, _DE), F32),
            jax.ShapeDtypeStruct((_N, _DE), F32),
            jax.ShapeDtypeStruct((1, _DG), F32),
            jax.ShapeDtypeStruct((1, _DE), F32),
        ],
        scratch_shapes=[pltpu.VMEM((1, _DN), F32), pltpu.VMEM((1, _DE), F32)],
    )

    p, q, ce, g = p0, q0, ce0, globals_
    for _step in range(3):
        ee = edge_pre(edges, We_e, ce)
        edges, recv2 = sc_step(ee, p, q, senders, receivers, zeros_pad)
        nodes, p, q, g, ce = node_step(
            nodes, recv2, g,
            Wn_n, Wn_r, Wn_g, wnb,
            We_s, We_r, We_g, web,
            Wg_n, Wg_e, Wg_g, wgb,
        )

    return (nodes, edges, g)


# speedup vs baseline: 4.6362x; 1.3900x over previous
import functools

import jax
import jax.numpy as jnp
from jax import lax
from jax.experimental import pallas as pl
from jax.experimental.pallas import tpu as pltpu
from jax.experimental.pallas import tpu_sc as plsc

F32 = jnp.float32

_N = 10000
_E = 320000
_DN = 128
_DE = 16
_DG = 8

_CHUNK = 128
_NCHUNKS = _E // _CHUNK
_K = 4
_GROUP = _K * _CHUNK
_NGROUPS = _NCHUNKS // _K
_NW = 32
_NPAD = 10240
_ROWS_PER_SUB = _NPAD // 16

_NBLK = 400
_EBLK = 2000


def _init_body(nodes_ref, g_ref, wes_ref, wer_ref, weg_ref, web_ref,
               p_ref, q_ref, ce_ref):
    nb = nodes_ref[...]
    p_ref[...] = jnp.dot(nb, wes_ref[...], preferred_element_type=F32)
    q_ref[...] = jnp.dot(nb, wer_ref[...], preferred_element_type=F32)

    @pl.when(pl.program_id(0) == 0)
    def _():
        ce_ref[...] = (
            jnp.dot(g_ref[...], weg_ref[...], preferred_element_type=F32)
            + web_ref[...]
        )


def _edge_pre_body(edges_ref, wee_ref, ce_ref, ee_ref):
    ee_ref[...] = (
        jnp.dot(edges_ref[...], wee_ref[...], preferred_element_type=F32)
        + ce_ref[...]
    )


def _sc_step_body(ee_hbm, p_hbm, q_hbm, sr_hbm, z_hbm,
                  eout_hbm, recv_hbm,
                  idx_v, ee_v, ps_v, qs_v, out_v, acc_sh, sem, gsem):
    cid = lax.axis_index("c")
    sid = lax.axis_index("s")
    wid = sid * 2 + cid

    row0 = sid * _ROWS_PER_SUB
    pltpu.sync_copy(z_hbm.at[pl.ds(row0, _ROWS_PER_SUB)],
                    acc_sh.at[pl.ds(row0, _ROWS_PER_SUB)])
    plsc.subcore_barrier()

    def group_body(j, carry):
        grp = wid + j * _NW
        base = grp * _GROUP
        pltpu.sync_copy(sr_hbm.at[pl.ds(grp * _K, _K)], idx_v)
        dee = pltpu.async_copy(ee_hbm.at[pl.ds(base, _GROUP)], ee_v, sem)
        gds = []
        for k in range(_K):
            gds.append(pltpu.async_copy(
                p_hbm.at[idx_v.at[k, 0]],
                ps_v.at[pl.ds(k * _CHUNK, _CHUNK)], gsem))
            gds.append(pltpu.async_copy(
                q_hbm.at[idx_v.at[k, 1]],
                qs_v.at[pl.ds(k * _CHUNK, _CHUNK)], gsem))
        dee.wait()
        for d in gds:
            d.wait()

        def row_body(i, a):
            b = i * 4
            for u in range(4):
                out_v[b + u, :] = jnp.maximum(
                    ee_v[b + u, :] + ps_v[b + u, :] + qs_v[b + u, :], 0.0)
            return a

        lax.fori_loop(0, _GROUP // 4, row_body, 0)
        pltpu.sync_copy(out_v, eout_hbm.at[pl.ds(base, _GROUP)])
        for k in range(_K):
            pltpu.sync_copy(out_v.at[pl.ds(k * _CHUNK, _CHUNK)],
                            acc_sh.at[idx_v.at[k, 1]], add=True)
        return carry

    nj = (_NGROUPS - wid + _NW - 1) // _NW
    lax.fori_loop(0, nj, group_body, 0)

    plsc.subcore_barrier()
    pltpu.sync_copy(acc_sh.at[pl.ds(row0, _ROWS_PER_SUB)],
                    recv_hbm.at[cid, pl.ds(row0, _ROWS_PER_SUB)])


def _node_body(nodes_ref, recv2_ref, g_ref,
               wnn_ref, wnr_ref, wng_ref, wnb_ref,
               wes_ref, wer_ref, weg_ref, web_ref,
               wgn_ref, wge_ref, wgg_ref, wgb_ref,
               nout_ref, p_ref, q_ref, gout_ref, ce_ref,
               accn_ref, acce_ref):
    i = pl.program_id(0)
    g = g_ref[...]
    c_n = jnp.dot(g, wng_ref[...], preferred_element_type=F32) + wnb_ref[...]
    recv = recv2_ref[0] + recv2_ref[1]
    out = (
        jnp.dot(nodes_ref[...], wnn_ref[...], preferred_element_type=F32)
        + jnp.dot(recv, wnr_ref[...], preferred_element_type=F32)
        + c_n
    )
    out = jnp.maximum(out, 0.0)
    nout_ref[...] = out
    p_ref[...] = jnp.dot(out, wes_ref[...], preferred_element_type=F32)
    q_ref[...] = jnp.dot(out, wer_ref[...], preferred_element_type=F32)

    @pl.when(i == 0)
    def _():
        accn_ref[...] = jnp.zeros_like(accn_ref)
        acce_ref[...] = jnp.zeros_like(acce_ref)

    accn_ref[...] += jnp.sum(out, axis=0, keepdims=True)
    acce_ref[...] += jnp.sum(recv, axis=0, keepdims=True)

    @pl.when(i == pl.num_programs(0) - 1)
    def _():
        agg_n = accn_ref[...]
        agg_e = acce_ref[...]
        g_new = (
            jnp.dot(agg_n, wgn_ref[...], preferred_element_type=F32)
            + jnp.dot(agg_e, wge_ref[...], preferred_element_type=F32)
            + jnp.dot(g, wgg_ref[...], preferred_element_type=F32)
            + wgb_ref[...]
        )
        gout_ref[...] = g_new
        ce_ref[...] = (
            jnp.dot(g_new, weg_ref[...], preferred_element_type=F32)
            + web_ref[...]
        )


def _full(i):
    return 0


def kernel(nodes, edges, globals_, senders, receivers,
           We_W, We_b, Wn_W, Wn_b, Wg_W, Wg_b):
    We_e = We_W[:_DE]
    We_s = We_W[_DE:_DE + _DN]
    We_r = We_W[_DE + _DN:_DE + 2 * _DN]
    We_g = We_W[_DE + 2 * _DN:]
    Wn_n = Wn_W[:_DN]
    Wn_r = Wn_W[_DN:_DN + _DE]
    Wn_g = Wn_W[_DN + _DE:]
    Wg_n = Wg_W[:_DN]
    Wg_e = Wg_W[_DN:_DN + _DE]
    Wg_g = Wg_W[_DN + _DE:]
    web = We_b.reshape(1, _DE)
    wnb = Wn_b.reshape(1, _DN)
    wgb = Wg_b.reshape(1, _DG)
    zeros_pad = jnp.zeros((_NPAD, _DE), F32)
    sr_packed = jnp.stack(
        [senders.reshape(_NCHUNKS, _CHUNK), receivers.reshape(_NCHUNKS, _CHUNK)],
        axis=1)

    n_grid = _N // _NBLK
    e_grid = _E // _EBLK

    p0, q0, ce0 = pl.pallas_call(
        _init_body,
        grid=(n_grid,),
        in_specs=[
            pl.BlockSpec((_NBLK, _DN), lambda i: (i, 0)),
            pl.BlockSpec((1, _DG), lambda i: (0, 0)),
            pl.BlockSpec((_DN, _DE), lambda i: (0, 0)),
            pl.BlockSpec((_DN, _DE), lambda i: (0, 0)),
            pl.BlockSpec((_DG, _DE), lambda i: (0, 0)),
            pl.BlockSpec((1, _DE), lambda i: (0, 0)),
        ],
        out_specs=[
            pl.BlockSpec((_NBLK, _DE), lambda i: (i, 0)),
            pl.BlockSpec((_NBLK, _DE), lambda i: (i, 0)),
            pl.BlockSpec((1, _DE), lambda i: (0, 0)),
        ],
        out_shape=[
            jax.ShapeDtypeStruct((_N, _DE), F32),
            jax.ShapeDtypeStruct((_N, _DE), F32),
            jax.ShapeDtypeStruct((1, _DE), F32),
        ],
    )(nodes, globals_, We_s, We_r, We_g, web)

    edge_pre = pl.pallas_call(
        _edge_pre_body,
        grid=(e_grid,),
        in_specs=[
            pl.BlockSpec((_EBLK, _DE), lambda i: (i, 0)),
            pl.BlockSpec((_DE, _DE), lambda i: (0, 0)),
            pl.BlockSpec((1, _DE), lambda i: (0, 0)),
        ],
        out_specs=pl.BlockSpec((_EBLK, _DE), lambda i: (i, 0)),
        out_shape=jax.ShapeDtypeStruct((_E, _DE), F32),
    )

    sc_step = pl.kernel(
        _sc_step_body,
        out_type=[
            jax.ShapeDtypeStruct((_E, _DE), F32),
            jax.ShapeDtypeStruct((2, _NPAD, _DE), F32),
        ],
        mesh=plsc.VectorSubcoreMesh(core_axis_name="c", subcore_axis_name="s"),
        compiler_params=pltpu.CompilerParams(use_tc_tiling_on_sc=False),
        scratch_types=[
            pltpu.VMEM((_K, 2, _CHUNK), jnp.int32),
            pltpu.VMEM((_GROUP, _DE), F32),
            pltpu.VMEM((_GROUP, _DE), F32),
            pltpu.VMEM((_GROUP, _DE), F32),
            pltpu.VMEM((_GROUP, _DE), F32),
            pltpu.VMEM_SHARED((_NPAD, _DE), F32),
            pltpu.SemaphoreType.DMA,
            pltpu.SemaphoreType.DMA,
        ],
    )

    node_step = pl.pallas_call(
        _node_body,
        grid=(n_grid,),
        in_specs=[
            pl.BlockSpec((_NBLK, _DN), lambda i: (i, 0)),
            pl.BlockSpec((2, _NBLK, _DE), lambda i: (0, i, 0)),
            pl.BlockSpec((1, _DG), lambda i: (0, 0)),
            pl.BlockSpec((_DN, _DN), lambda i: (0, 0)),
            pl.BlockSpec((_DE, _DN), lambda i: (0, 0)),
            pl.BlockSpec((_DG, _DN), lambda i: (0, 0)),
            pl.BlockSpec((1, _DN), lambda i: (0, 0)),
            pl.BlockSpec((_DN, _DE), lambda i: (0, 0)),
            pl.BlockSpec((_DN, _DE), lambda i: (0, 0)),
            pl.BlockSpec((_DG, _DE), lambda i: (0, 0)),
            pl.BlockSpec((1, _DE), lambda i: (0, 0)),
            pl.BlockSpec((_DN, _DG), lambda i: (0, 0)),
            pl.BlockSpec((_DE, _DG), lambda i: (0, 0)),
            pl.BlockSpec((_DG, _DG), lambda i: (0, 0)),
            pl.BlockSpec((1, _DG), lambda i: (0, 0)),
        ],
        out_specs=[
            pl.BlockSpec((_NBLK, _DN), lambda i: (i, 0)),
            pl.BlockSpec((_NBLK, _DE), lambda i: (i, 0)),
            pl.BlockSpec((_NBLK, _DE), lambda i: (i, 0)),
            pl.BlockSpec((1, _DG), lambda i: (0, 0)),
            pl.BlockSpec((1, _DE), lambda i: (0, 0)),
        ],
        out_shape=[
            jax.ShapeDtypeStruct((_N, _DN), F32),
            jax.ShapeDtypeStruct((_N, _DE), F32),
            jax.ShapeDtypeStruct((_N, _DE), F32),
            jax.ShapeDtypeStruct((1, _DG), F32),
            jax.ShapeDtypeStruct((1, _DE), F32),
        ],
        scratch_shapes=[pltpu.VMEM((1, _DN), F32), pltpu.VMEM((1, _DE), F32)],
    )

    p, q, ce, g = p0, q0, ce0, globals_
    for _step in range(3):
        ee = edge_pre(edges, We_e, ce)
        edges, recv2 = sc_step(ee, p, q, sr_packed, zeros_pad)
        nodes, p, q, g, ce = node_step(
            nodes, recv2, g,
            Wn_n, Wn_r, Wn_g, wnb,
            We_s, We_r, We_g, web,
            Wg_n, Wg_e, Wg_g, wgb,
        )

    return (nodes, edges, g)


# speedup vs baseline: 10.3273x; 2.2275x over previous
import functools

import jax
import jax.numpy as jnp
from jax import lax
from jax.experimental import pallas as pl
from jax.experimental.pallas import tpu as pltpu
from jax.experimental.pallas import tpu_sc as plsc

F32 = jnp.float32

_N = 10000
_E = 320000
_DN = 128
_DE = 16
_DG = 8

_CHUNK = 128
_NCHUNKS = _E // _CHUNK
_K = 4
_GROUP = _K * _CHUNK
_NGROUPS = _NCHUNKS // _K
_NW = 32
_NPAD = 10240
_ROWS_PER_SUB = _NPAD // 16

_NBLK = 400
_E8 = _E // 8
_EBLK8 = 2000


def _init_body(nodes_ref, g_ref, wes_ref, wer_ref, weg_ref, web_ref,
               p_ref, q_ref, ce_ref):
    nb = nodes_ref[...]
    p_ref[...] = jnp.dot(nb, wes_ref[...], preferred_element_type=F32)
    q_ref[...] = jnp.dot(nb, wer_ref[...], preferred_element_type=F32)

    @pl.when(pl.program_id(0) == 0)
    def _():
        ce_ref[...] = (
            jnp.dot(g_ref[...], weg_ref[...], preferred_element_type=F32)
            + web_ref[...]
        )


def _edge_pre_body(edges_ref, wbd_ref, cet_ref, ee_ref):
    ee_ref[...] = (
        jnp.dot(edges_ref[...], wbd_ref[...], preferred_element_type=F32)
        + cet_ref[...]
    )


def _sc_step_body(ee_hbm, p_hbm, q_hbm, sr_hbm, z_hbm,
                  eout_hbm, recv_hbm,
                  idx_v, ee_v, ps_v, qs_v, out_v, acc_sh, sem, gsem):
    cid = lax.axis_index("c")
    sid = lax.axis_index("s")
    wid = sid * 2 + cid

    row0 = sid * _ROWS_PER_SUB
    pltpu.sync_copy(z_hbm.at[pl.ds(row0, _ROWS_PER_SUB)],
                    acc_sh.at[pl.ds(row0, _ROWS_PER_SUB)])
    plsc.subcore_barrier()

    def group_body(j, carry):
        grp = wid + j * _NW
        base = grp * _GROUP
        pltpu.sync_copy(sr_hbm.at[pl.ds(grp * _K, _K)], idx_v)
        dee = pltpu.async_copy(ee_hbm.at[pl.ds(base, _GROUP)], ee_v, sem)
        gds = []
        for k in range(_K):
            gds.append(pltpu.async_copy(
                p_hbm.at[idx_v.at[k, 0]],
                ps_v.at[pl.ds(k * _CHUNK, _CHUNK)], gsem))
            gds.append(pltpu.async_copy(
                q_hbm.at[idx_v.at[k, 1]],
                qs_v.at[pl.ds(k * _CHUNK, _CHUNK)], gsem))
        dee.wait()
        for d in gds:
            d.wait()

        def row_body(i, a):
            b = i * 4
            for u in range(4):
                out_v[b + u, :] = jnp.maximum(
                    ee_v[b + u, :] + ps_v[b + u, :] + qs_v[b + u, :], 0.0)
            return a

        lax.fori_loop(0, _GROUP // 4, row_body, 0)
        pltpu.sync_copy(out_v, eout_hbm.at[pl.ds(base, _GROUP)])
        for k in range(_K):
            pltpu.sync_copy(out_v.at[pl.ds(k * _CHUNK, _CHUNK)],
                            acc_sh.at[idx_v.at[k, 1]], add=True)
        return carry

    nj = (_NGROUPS - wid + _NW - 1) // _NW
    lax.fori_loop(0, nj, group_body, 0)

    plsc.subcore_barrier()
    pltpu.sync_copy(acc_sh.at[pl.ds(row0, _ROWS_PER_SUB)],
                    recv_hbm.at[cid, pl.ds(row0, _ROWS_PER_SUB)])


def _node_body(nodes_ref, recv2_ref, g_ref,
               wnn_ref, wnr_ref, wng_ref, wnb_ref,
               wes_ref, wer_ref, weg_ref, web_ref,
               wgn_ref, wge_ref, wgg_ref, wgb_ref,
               nout_ref, p_ref, q_ref, gout_ref, ce_ref,
               accn_ref, acce_ref):
    i = pl.program_id(0)
    g = g_ref[...]
    c_n = jnp.dot(g, wng_ref[...], preferred_element_type=F32) + wnb_ref[...]
    recv = recv2_ref[0] + recv2_ref[1]
    out = (
        jnp.dot(nodes_ref[...], wnn_ref[...], preferred_element_type=F32)
        + jnp.dot(recv, wnr_ref[...], preferred_element_type=F32)
        + c_n
    )
    out = jnp.maximum(out, 0.0)
    nout_ref[...] = out
    p_ref[...] = jnp.dot(out, wes_ref[...], preferred_element_type=F32)
    q_ref[...] = jnp.dot(out, wer_ref[...], preferred_element_type=F32)

    @pl.when(i == 0)
    def _():
        accn_ref[...] = jnp.zeros_like(accn_ref)
        acce_ref[...] = jnp.zeros_like(acce_ref)

    accn_ref[...] += jnp.sum(out, axis=0, keepdims=True)
    acce_ref[...] += jnp.sum(recv, axis=0, keepdims=True)

    @pl.when(i == pl.num_programs(0) - 1)
    def _():
        agg_n = accn_ref[...]
        agg_e = acce_ref[...]
        g_new = (
            jnp.dot(agg_n, wgn_ref[...], preferred_element_type=F32)
            + jnp.dot(agg_e, wge_ref[...], preferred_element_type=F32)
            + jnp.dot(g, wgg_ref[...], preferred_element_type=F32)
            + wgb_ref[...]
        )
        gout_ref[...] = g_new
        ce_ref[...] = (
            jnp.dot(g_new, weg_ref[...], preferred_element_type=F32)
            + web_ref[...]
        )


def _full(i):
    return 0


def kernel(nodes, edges, globals_, senders, receivers,
           We_W, We_b, Wn_W, Wn_b, Wg_W, Wg_b):
    We_e = We_W[:_DE]
    We_s = We_W[_DE:_DE + _DN]
    We_r = We_W[_DE + _DN:_DE + 2 * _DN]
    We_g = We_W[_DE + 2 * _DN:]
    Wn_n = Wn_W[:_DN]
    Wn_r = Wn_W[_DN:_DN + _DE]
    Wn_g = Wn_W[_DN + _DE:]
    Wg_n = Wg_W[:_DN]
    Wg_e = Wg_W[_DN:_DN + _DE]
    Wg_g = Wg_W[_DN + _DE:]
    web = We_b.reshape(1, _DE)
    wnb = Wn_b.reshape(1, _DN)
    wgb = Wg_b.reshape(1, _DG)
    zeros_pad = jnp.zeros((_NPAD, _DE), F32)
    sr_packed = jnp.stack(
        [senders.reshape(_NCHUNKS, _CHUNK), receivers.reshape(_NCHUNKS, _CHUNK)],
        axis=1)

    n_grid = _N // _NBLK
    e_grid = _E8 // _EBLK8

    p0, q0, ce0 = pl.pallas_call(
        _init_body,
        grid=(n_grid,),
        in_specs=[
            pl.BlockSpec((_NBLK, _DN), lambda i: (i, 0)),
            pl.BlockSpec((1, _DG), lambda i: (0, 0)),
            pl.BlockSpec((_DN, _DE), lambda i: (0, 0)),
            pl.BlockSpec((_DN, _DE), lambda i: (0, 0)),
            pl.BlockSpec((_DG, _DE), lambda i: (0, 0)),
            pl.BlockSpec((1, _DE), lambda i: (0, 0)),
        ],
        out_specs=[
            pl.BlockSpec((_NBLK, _DE), lambda i: (i, 0)),
            pl.BlockSpec((_NBLK, _DE), lambda i: (i, 0)),
            pl.BlockSpec((1, _DE), lambda i: (0, 0)),
        ],
        out_shape=[
            jax.ShapeDtypeStruct((_N, _DE), F32),
            jax.ShapeDtypeStruct((_N, _DE), F32),
            jax.ShapeDtypeStruct((1, _DE), F32),
        ],
    )(nodes, globals_, We_s, We_r, We_g, web)

    edge_pre = pl.pallas_call(
        _edge_pre_body,
        grid=(e_grid,),
        in_specs=[
            pl.BlockSpec((_EBLK8, 128), lambda i: (i, 0)),
            pl.BlockSpec((128, 128), lambda i: (0, 0)),
            pl.BlockSpec((1, 128), lambda i: (0, 0)),
        ],
        out_specs=pl.BlockSpec((_EBLK8, 128), lambda i: (i, 0)),
        out_shape=jax.ShapeDtypeStruct((_E8, 128), F32),
    )

    sc_step = pl.kernel(
        _sc_step_body,
        out_type=[
            jax.ShapeDtypeStruct((_E, _DE), F32),
            jax.ShapeDtypeStruct((2, _NPAD, _DE), F32),
        ],
        mesh=plsc.VectorSubcoreMesh(core_axis_name="c", subcore_axis_name="s"),
        compiler_params=pltpu.CompilerParams(use_tc_tiling_on_sc=False),
        scratch_types=[
            pltpu.VMEM((_K, 2, _CHUNK), jnp.int32),
            pltpu.VMEM((_GROUP, _DE), F32),
            pltpu.VMEM((_GROUP, _DE), F32),
            pltpu.VMEM((_GROUP, _DE), F32),
            pltpu.VMEM((_GROUP, _DE), F32),
            pltpu.VMEM_SHARED((_NPAD, _DE), F32),
            pltpu.SemaphoreType.DMA,
            pltpu.SemaphoreType.DMA,
        ],
    )

    node_step = pl.pallas_call(
        _node_body,
        grid=(n_grid,),
        in_specs=[
            pl.BlockSpec((_NBLK, _DN), lambda i: (i, 0)),
            pl.BlockSpec((2, _NBLK, _DE), lambda i: (0, i, 0)),
            pl.BlockSpec((1, _DG), lambda i: (0, 0)),
            pl.BlockSpec((_DN, _DN), lambda i: (0, 0)),
            pl.BlockSpec((_DE, _DN), lambda i: (0, 0)),
            pl.BlockSpec((_DG, _DN), lambda i: (0, 0)),
            pl.BlockSpec((1, _DN), lambda i: (0, 0)),
            pl.BlockSpec((_DN, _DE), lambda i: (0, 0)),
            pl.BlockSpec((_DN, _DE), lambda i: (0, 0)),
            pl.BlockSpec((_DG, _DE), lambda i: (0, 0)),
            pl.BlockSpec((1, _DE), lambda i: (0, 0)),
            pl.BlockSpec((_DN, _DG), lambda i: (0, 0)),
            pl.BlockSpec((_DE, _DG), lambda i: (0, 0)),
            pl.BlockSpec((_DG, _DG), lambda i: (0, 0)),
            pl.BlockSpec((1, _DG), lambda i: (0, 0)),
        ],
        out_specs=[
            pl.BlockSpec((_NBLK, _DN), lambda i: (i, 0)),
            pl.BlockSpec((_NBLK, _DE), lambda i: (i, 0)),
            pl.BlockSpec((_NBLK, _DE), lambda i: (i, 0)),
            pl.BlockSpec((1, _DG), lambda i: (0, 0)),
            pl.BlockSpec((1, _DE), lambda i: (0, 0)),
        ],
        out_shape=[
            jax.ShapeDtypeStruct((_N, _DN), F32),
            jax.ShapeDtypeStruct((_N, _DE), F32),
            jax.ShapeDtypeStruct((_N, _DE), F32),
            jax.ShapeDtypeStruct((1, _DG), F32),
            jax.ShapeDtypeStruct((1, _DE), F32),
        ],
        scratch_shapes=[pltpu.VMEM((1, _DN), F32), pltpu.VMEM((1, _DE), F32)],
    )

    W_bd = jnp.kron(jnp.eye(8, dtype=F32), We_e)
    edges_p = edges.reshape(_E8, 128)
    p, q, ce, g = p0, q0, ce0, globals_
    for _step in range(3):
        ee_p = edge_pre(edges_p, W_bd, jnp.tile(ce, (1, 8)))
        edges_lin, recv2 = sc_step(
            ee_p.reshape(_E, _DE), p, q, sr_packed, zeros_pad)
        edges_p = edges_lin.reshape(_E8, 128)
        nodes, p, q, g, ce = node_step(
            nodes, recv2, g,
            Wn_n, Wn_r, Wn_g, wnb,
            We_s, We_r, We_g, web,
            Wg_n, Wg_e, Wg_g, wgb,
        )

    return (nodes, edges_lin, g)
